# Initial kernel scaffold; baseline (speedup 1.0000x reference)
import jax
import jax.numpy as jnp
from jax.experimental import pallas as pl


def kernel(x, edge_index, edge_weight, W, a_src, a_dst):
    raise NotImplementedError("write your pallas kernel here")


# speedup vs baseline: 43.8518x; 43.8518x over previous
import jax
import jax.numpy as jnp
from jax import lax
from jax.experimental import pallas as pl
from jax.experimental.pallas import tpu as pltpu
from jax.experimental.pallas import tpu_sc as plsc

HEADS = 4
OUT_CH = 32
IN_CH = 128
N_NODES = 10000
N_EDGES = 320000
HC = HEADS * OUT_CH

NC = 2
NS = 16
L = 16
HPC = HEADS // NC

CHUNK = 80
NG = CHUNK // L
ROWS = N_EDGES // CHUNK

ROWS_PER_TILE = N_EDGES // NS // CHUNK
BLOCKS = 10
ROWS_PER_BLOCK = ROWS_PER_TILE // BLOCKS

NODE_SLICE = N_NODES // NS


def _sc_body(h4_hbm, s8_hbm, src_hbm, dst_hbm, ew_hbm, zo_hbm, zt_hbm,
             outp_hbm, s8_t, srcblk, dstblk, ewblk, srcadj, erows, hbuf,
             tbuf, t_sp, out_sp):
    cid = lax.axis_index("c")
    sid = lax.axis_index("s")

    pltpu.sync_copy(s8_hbm, s8_t)
    pltpu.sync_copy(zo_hbm, out_sp.at[pl.ds(sid * NODE_SLICE, NODE_SLICE)])
    pltpu.sync_copy(zt_hbm, t_sp.at[pl.ds(sid * NODE_SLICE, NODE_SLICE)])
    pltpu.sync_copy(zt_hbm.at[pl.ds(0, CHUNK)], erows)
    plsc.subcore_barrier()

    def stage_block(b):
        base_row = sid * ROWS_PER_TILE + b * ROWS_PER_BLOCK
        pltpu.sync_copy(
            src_hbm.at[pl.ds(base_row, ROWS_PER_BLOCK)], srcblk)
        pltpu.sync_copy(dst_hbm.at[pl.ds(base_row, ROWS_PER_BLOCK)], dstblk)
        pltpu.sync_copy(ew_hbm.at[pl.ds(base_row, ROWS_PER_BLOCK)], ewblk)

    def edge_alpha_num(k, g, hl):
        sidx = srcblk[k, pl.ds(g * L, L)]
        didx = dstblk[k, pl.ds(g * L, L)]
        w = ewblk[k, pl.ds(g * L, L)]
        hsplat = jnp.full((L,), hl, jnp.int32) + cid * HPC
        gs = plsc.load_gather(s8_t, [sidx, hsplat])
        gd = plsc.load_gather(s8_t, [didx, hsplat + HEADS])
        r = gs + gd
        r = jnp.where(r >= 0.0, r, r * 0.2) * w
        return jnp.exp(r)

    def p1_block(b, carry):
        stage_block(b)

        def p1_chunk(k, c2):
            for g in range(NG):
                lane = lax.iota(jnp.int32, L) + g * L
                for hl in range(HPC):
                    ev = edge_alpha_num(k, g, hl)
                    plsc.store_scatter(
                        erows, [lane, jnp.full((L,), hl, jnp.int32)], ev)
            pltpu.sync_copy(erows, t_sp.at[dstblk.at[k]], add=True)
            return c2

        return lax.fori_loop(0, ROWS_PER_BLOCK, p1_chunk, carry)

    lax.fori_loop(0, BLOCKS, p1_block, 0)
    plsc.subcore_barrier()

    def p2_subphase(hl):
        row_off = (cid * HPC + hl) * N_NODES

        def p2_block(b, carry):
            stage_block(b)

            def p2_chunk(k, c2):
                for g in range(NG):
                    srcadj[pl.ds(g * L, L)] = (
                        srcblk[k, pl.ds(g * L, L)] + row_off)
                pltpu.sync_copy(h4_hbm.at[srcadj], hbuf)
                pltpu.sync_copy(t_sp.at[dstblk.at[k]], tbuf)
                for g in range(NG):
                    lane = lax.iota(jnp.int32, L) + g * L
                    ev = edge_alpha_num(k, g, hl)
                    tg = plsc.load_gather(
                        tbuf, [lane, jnp.full((L,), hl, jnp.int32)])
                    a = ev / (tg + 1e-8)
                    for e16 in range(L):
                        e = g * L + e16
                        aa = a[e16]
                        hbuf[e, pl.ds(0, L)] = hbuf[e, pl.ds(0, L)] * aa
                        hbuf[e, pl.ds(L, L)] = hbuf[e, pl.ds(L, L)] * aa
                pltpu.sync_copy(hbuf, out_sp.at[dstblk.at[k]], add=True)
                return c2

            return lax.fori_loop(0, ROWS_PER_BLOCK, p2_chunk, carry)

        lax.fori_loop(0, BLOCKS, p2_block, 0)
        plsc.subcore_barrier()
        pltpu.sync_copy(
            out_sp.at[pl.ds(sid * NODE_SLICE, NODE_SLICE)],
            outp_hbm.at[cid, hl, pl.ds(sid * NODE_SLICE, NODE_SLICE)])
        if hl + 1 < HPC:
            pltpu.sync_copy(
                zo_hbm, out_sp.at[pl.ds(sid * NODE_SLICE, NODE_SLICE)])
            plsc.subcore_barrier()

    for hl in range(HPC):
        p2_subphase(hl)


_sc_edge_phase = pl.kernel(
    _sc_body,
    out_type=jax.ShapeDtypeStruct((NC, HPC, N_NODES, OUT_CH), jnp.float32),
    mesh=plsc.VectorSubcoreMesh(core_axis_name="c", subcore_axis_name="s"),
    compiler_params=pltpu.CompilerParams(
        use_tc_tiling_on_sc=False, needs_layout_passes=False),
    scratch_types=[
        pltpu.VMEM((N_NODES, 2 * HEADS), jnp.float32),
        pltpu.VMEM((ROWS_PER_BLOCK, CHUNK), jnp.int32),
        pltpu.VMEM((ROWS_PER_BLOCK, CHUNK), jnp.int32),
        pltpu.VMEM((ROWS_PER_BLOCK, CHUNK), jnp.float32),
        pltpu.VMEM((CHUNK,), jnp.int32),
        pltpu.VMEM((CHUNK, 16), jnp.float32),
        pltpu.VMEM((CHUNK, OUT_CH), jnp.float32),
        pltpu.VMEM((CHUNK, 16), jnp.float32),
        pltpu.VMEM_SHARED((N_NODES, 16), jnp.float32),
        pltpu.VMEM_SHARED((N_NODES, OUT_CH), jnp.float32),
    ],
)


def _proj_body(x_ref, wt_ref, wa_ref, h_ref, s_ref):
    h_ref[...] = jnp.dot(x_ref[...], wt_ref[0],
                         preferred_element_type=jnp.float32)
    s_ref[...] = jnp.dot(x_ref[...], wa_ref[...],
                         preferred_element_type=jnp.float32)


_PROJ_BLK = 2000


def kernel(x, edge_index, edge_weight, W, a_src, a_dst):
    src = edge_index[0].astype(jnp.int32).reshape(ROWS, CHUNK)
    dst = edge_index[1].astype(jnp.int32).reshape(ROWS, CHUNK)
    ew = edge_weight.reshape(ROWS, CHUNK)
    wt = W.T.astype(jnp.float32)

    k = jnp.arange(HC)
    head_mask = (k[:, None] // OUT_CH == jnp.arange(HEADS)[None, :])
    a_mat = jnp.concatenate(
        [jnp.where(head_mask, a_src.reshape(-1)[:, None], 0.0),
         jnp.where(head_mask, a_dst.reshape(-1)[:, None], 0.0)],
        axis=1).astype(jnp.float32)
    wa = wt @ a_mat

    zo = jnp.zeros((NODE_SLICE, OUT_CH), jnp.float32)
    zt = jnp.zeros((NODE_SLICE, 16), jnp.float32)

    n_blocks = N_NODES // _PROJ_BLK
    h4, s8 = pl.pallas_call(
        _proj_body,
        grid=(HEADS, n_blocks),
        in_specs=[
            pl.BlockSpec((_PROJ_BLK, IN_CH), lambda hh, i: (i, 0)),
            pl.BlockSpec((1, IN_CH, OUT_CH), lambda hh, i: (hh, 0, 0)),
            pl.BlockSpec((IN_CH, 2 * HEADS), lambda hh, i: (0, 0)),
        ],
        out_specs=[
            pl.BlockSpec((_PROJ_BLK, OUT_CH),
                         lambda hh, i: (hh * (N_NODES // _PROJ_BLK) + i, 0)),
            pl.BlockSpec((_PROJ_BLK, 2 * HEADS), lambda hh, i: (i, 0)),
        ],
        out_shape=[
            jax.ShapeDtypeStruct((HEADS * N_NODES, OUT_CH), jnp.float32),
            jax.ShapeDtypeStruct((N_NODES, 2 * HEADS), jnp.float32),
        ],
    )(x, wt.reshape(IN_CH, HEADS, OUT_CH).transpose(1, 0, 2), wa)

    outp = _sc_edge_phase(h4, s8, src, dst, ew, zo, zt)
    return jnp.moveaxis(outp.reshape(HEADS, N_NODES, OUT_CH), 0, 1).reshape(
        N_NODES, HC)


# speedup vs baseline: 44.5006x; 1.0148x over previous
import jax
import jax.numpy as jnp
from jax import lax
from jax.experimental import pallas as pl
from jax.experimental.pallas import tpu as pltpu
from jax.experimental.pallas import tpu_sc as plsc

HEADS = 4
OUT_CH = 32
IN_CH = 128
N_NODES = 10000
N_EDGES = 320000
HC = HEADS * OUT_CH

NC = 2
NS = 16
L = 16
HPC = HEADS // NC
TW = 16

CHUNK = 80
NG = CHUNK // L
ROWS = N_EDGES // CHUNK

ROWS_PER_TILE = N_EDGES // NS // CHUNK
BLOCKS = 5
RPB = ROWS_PER_TILE // BLOCKS

NODE_SLICE = N_NODES // NS


def _sc_body(h4_hbm, s8_hbm, edges_hbm, zo_hbm, zt_hbm,
             outp_hbm, s8_t, eb,
             srcadj0, srcadj1, erows0, erows1, hbuf0, hbuf1, tbuf0, tbuf1,
             t_sp, out_sp, gsem0, gsem1, ssem0, ssem1):
    cid = lax.axis_index("c")
    sid = lax.axis_index("s")
    gsem = [gsem0, gsem1]
    ssem = [ssem0, ssem1]
    erows = [erows0, erows1]
    hbuf = [hbuf0, hbuf1]
    tbuf = [tbuf0, tbuf1]
    srcadj = [srcadj0, srcadj1]

    pltpu.sync_copy(s8_hbm, s8_t)
    pltpu.sync_copy(zo_hbm, out_sp.at[pl.ds(sid * NODE_SLICE, NODE_SLICE)])
    pltpu.sync_copy(zt_hbm, t_sp.at[pl.ds(sid * NODE_SLICE, NODE_SLICE)])
    pltpu.sync_copy(zt_hbm.at[pl.ds(0, CHUNK)], erows0)
    pltpu.sync_copy(zt_hbm.at[pl.ds(0, CHUNK)], erows1)
    plsc.subcore_barrier()

    def stage(b):
        base = sid * ROWS_PER_TILE + b * RPB
        pltpu.sync_copy(edges_hbm.at[pl.ds(base, RPB)], eb)

    def edge_vecs(k, g):
        sidx = eb[k, 0, pl.ds(g * L, L)]
        didx = eb[k, 1, pl.ds(g * L, L)]
        w = plsc.bitcast(eb[k, 2, pl.ds(g * L, L)], jnp.float32)
        return sidx, didx, w

    def edge_alpha_num(k, g, hl):
        sidx, didx, w = edge_vecs(k, g)
        hsplat = jnp.full((L,), 0, jnp.int32) + (hl + cid * HPC)
        gs = plsc.load_gather(s8_t, [sidx, hsplat])
        gd = plsc.load_gather(s8_t, [didx, hsplat + HEADS])
        r = gs + gd
        r = jnp.where(r >= 0.0, r, r * 0.2) * w
        return jnp.exp(r)

    def p1_compute(k, er):
        for g in range(NG):
            lane = lax.iota(jnp.int32, L) + g * L
            for hl in range(HPC):
                ev = edge_alpha_num(k, g, hl)
                plsc.store_scatter(
                    er, [lane, jnp.full((L,), hl, jnp.int32)], ev)

    def p1_block(b, carry):
        stage(b)

        def p1_chunk(k, c2):
            p1_compute(k, erows0)
            pltpu.sync_copy(erows0, t_sp.at[eb.at[k, 1]], add=True)
            return c2

        lax.fori_loop(0, RPB, p1_chunk, 0)
        return carry

    lax.fori_loop(0, BLOCKS, p1_block, 0)
    plsc.subcore_barrier()

    def p2_subphase(hl):
        row_off = (cid * HPC + hl) * N_NODES

        def issue_gather(k, po):
            for g in range(NG):
                srcadj[po][pl.ds(g * L, L)] = (
                    eb[k, 0, pl.ds(g * L, L)] + row_off)
            pltpu.sync_copy(h4_hbm.at[srcadj[po]], hbuf[po])
            pltpu.sync_copy(t_sp.at[eb.at[k, 1]], tbuf[po])

        def p2_process(k, po):
            hb = hbuf[po]
            for g in range(NG):
                lane = lax.iota(jnp.int32, L) + g * L
                ev = edge_alpha_num(k, g, hl)
                tg = plsc.load_gather(
                    tbuf[po], [lane, jnp.full((L,), 0, jnp.int32) + hl])
                a = ev / (tg + 1e-8)
                for e16 in range(L):
                    e = g * L + e16
                    aa = a[e16]
                    hb[e, pl.ds(0, L)] = hb[e, pl.ds(0, L)] * aa
                    hb[e, pl.ds(L, L)] = hb[e, pl.ds(L, L)] * aa
            pltpu.sync_copy(hb, out_sp.at[eb.at[k, 1]], add=True)

        def p2_block(b, c2):
            stage(b)

            def p2_chunk(k, c3):
                issue_gather(k, 0)
                p2_process(k, 0)
                return c3

            lax.fori_loop(0, RPB, p2_chunk, 0)
            return c2

        lax.fori_loop(0, BLOCKS, p2_block, 0)
        plsc.subcore_barrier()
        pltpu.sync_copy(
            out_sp.at[pl.ds(sid * NODE_SLICE, NODE_SLICE)],
            outp_hbm.at[cid, hl, pl.ds(sid * NODE_SLICE, NODE_SLICE)])
        if hl + 1 < HPC:
            pltpu.sync_copy(
                zo_hbm, out_sp.at[pl.ds(sid * NODE_SLICE, NODE_SLICE)])
            plsc.subcore_barrier()

    for hl in range(HPC):
        p2_subphase(hl)


_sc_edge_phase = pl.kernel(
    _sc_body,
    out_type=jax.ShapeDtypeStruct((NC, HPC, N_NODES, OUT_CH), jnp.float32),
    mesh=plsc.VectorSubcoreMesh(core_axis_name="c", subcore_axis_name="s"),
    compiler_params=pltpu.CompilerParams(
        use_tc_tiling_on_sc=False, needs_layout_passes=False),
    scratch_types=[
        pltpu.VMEM((N_NODES, 2 * HEADS), jnp.float32),
        pltpu.VMEM((RPB, 3, CHUNK), jnp.int32),
        pltpu.VMEM((CHUNK,), jnp.int32),
        pltpu.VMEM((CHUNK,), jnp.int32),
        pltpu.VMEM((CHUNK, TW), jnp.float32),
        pltpu.VMEM((CHUNK, TW), jnp.float32),
        pltpu.VMEM((CHUNK, OUT_CH), jnp.float32),
        pltpu.VMEM((CHUNK, OUT_CH), jnp.float32),
        pltpu.VMEM((CHUNK, TW), jnp.float32),
        pltpu.VMEM((CHUNK, TW), jnp.float32),
        pltpu.VMEM_SHARED((N_NODES, TW), jnp.float32),
        pltpu.VMEM_SHARED((N_NODES, OUT_CH), jnp.float32),
        pltpu.SemaphoreType.DMA,
        pltpu.SemaphoreType.DMA,
        pltpu.SemaphoreType.DMA,
        pltpu.SemaphoreType.DMA,
    ],
)


def _proj_body(x_ref, wt_ref, wa_ref, h_ref, s_ref):
    h_ref[...] = jnp.dot(x_ref[...], wt_ref[0],
                         preferred_element_type=jnp.float32)
    s_ref[...] = jnp.dot(x_ref[...], wa_ref[...],
                         preferred_element_type=jnp.float32)


_PROJ_BLK = 2000


def kernel(x, edge_index, edge_weight, W, a_src, a_dst):
    src = edge_index[0].astype(jnp.int32).reshape(ROWS, CHUNK)
    dst = edge_index[1].astype(jnp.int32).reshape(ROWS, CHUNK)
    ewb = lax.bitcast_convert_type(
        edge_weight.astype(jnp.float32), jnp.int32).reshape(ROWS, CHUNK)
    edges = jnp.stack([src, dst, ewb], axis=1)
    wt = W.T.astype(jnp.float32)

    k = jnp.arange(HC)
    head_mask = (k[:, None] // OUT_CH == jnp.arange(HEADS)[None, :])
    a_mat = jnp.concatenate(
        [jnp.where(head_mask, a_src.reshape(-1)[:, None], 0.0),
         jnp.where(head_mask, a_dst.reshape(-1)[:, None], 0.0)],
        axis=1).astype(jnp.float32)
    wa = wt @ a_mat

    zo = jnp.zeros((NODE_SLICE, OUT_CH), jnp.float32)
    zt = jnp.zeros((NODE_SLICE, TW), jnp.float32)

    n_blocks = N_NODES // _PROJ_BLK
    h4, s8 = pl.pallas_call(
        _proj_body,
        grid=(HEADS, n_blocks),
        in_specs=[
            pl.BlockSpec((_PROJ_BLK, IN_CH), lambda hh, i: (i, 0)),
            pl.BlockSpec((1, IN_CH, OUT_CH), lambda hh, i: (hh, 0, 0)),
            pl.BlockSpec((IN_CH, 2 * HEADS), lambda hh, i: (0, 0)),
        ],
        out_specs=[
            pl.BlockSpec((_PROJ_BLK, OUT_CH),
                         lambda hh, i: (hh * (N_NODES // _PROJ_BLK) + i, 0)),
            pl.BlockSpec((_PROJ_BLK, 2 * HEADS), lambda hh, i: (i, 0)),
        ],
        out_shape=[
            jax.ShapeDtypeStruct((HEADS * N_NODES, OUT_CH), jnp.float32),
            jax.ShapeDtypeStruct((N_NODES, 2 * HEADS), jnp.float32),
        ],
    )(x, wt.reshape(IN_CH, HEADS, OUT_CH).transpose(1, 0, 2), wa)

    outp = _sc_edge_phase(h4, s8, edges, zo, zt)
    return jnp.moveaxis(outp.reshape(HEADS, N_NODES, OUT_CH), 0, 1).reshape(
        N_NODES, HC)


# speedup vs baseline: 59.3632x; 1.3340x over previous
import jax
import jax.numpy as jnp
from jax import lax
from jax.experimental import pallas as pl
from jax.experimental.pallas import tpu as pltpu
from jax.experimental.pallas import tpu_sc as plsc

HEADS = 4
OUT_CH = 32
IN_CH = 128
N_NODES = 10000
N_EDGES = 320000
HC = HEADS * OUT_CH

NC = 2
NS = 16
L = 16
HPC = HEADS // NC
TW = 8

CHUNK = 80
NG = CHUNK // L
ROWS = N_EDGES // CHUNK

ROWS_PER_TILE = N_EDGES // NS // CHUNK
BLOCKS = 5
RPB = ROWS_PER_TILE // BLOCKS

NODE_SLICE = N_NODES // NS


def _sc_body(h4_hbm, s8_hbm, edges_hbm, zo_hbm, zt_hbm,
             outp_hbm, s8_t, eb,
             srcadj0, srcadj1, erows0, erows1, hbuf0, hbuf1, tbuf0, tbuf1,
             t_sp, out_sp, gsem0, gsem1, ssem0, ssem1):
    cid = lax.axis_index("c")
    sid = lax.axis_index("s")
    gsem = [gsem0, gsem1]
    ssem = [ssem0, ssem1]
    erows = [erows0, erows1]
    hbuf = [hbuf0, hbuf1]
    tbuf = [tbuf0, tbuf1]
    srcadj = [srcadj0, srcadj1]

    pltpu.sync_copy(s8_hbm, s8_t)
    pltpu.sync_copy(zo_hbm, out_sp.at[pl.ds(sid * NODE_SLICE, NODE_SLICE)])
    pltpu.sync_copy(zt_hbm, t_sp.at[pl.ds(sid * NODE_SLICE, NODE_SLICE)])
    pltpu.sync_copy(zt_hbm.at[pl.ds(0, CHUNK)], erows0)
    pltpu.sync_copy(zt_hbm.at[pl.ds(0, CHUNK)], erows1)
    plsc.subcore_barrier()

    def stage(b):
        base = sid * ROWS_PER_TILE + b * RPB
        pltpu.sync_copy(edges_hbm.at[pl.ds(base, RPB)], eb)

    def edge_vecs(k, g):
        sidx = eb[k, 0, pl.ds(g * L, L)]
        didx = eb[k, 1, pl.ds(g * L, L)]
        w = plsc.bitcast(eb[k, 2, pl.ds(g * L, L)], jnp.float32)
        return sidx, didx, w

    def edge_alpha_num(k, g, hl):
        sidx, didx, w = edge_vecs(k, g)
        hsplat = jnp.full((L,), 0, jnp.int32) + (hl + cid * HPC)
        gs = plsc.load_gather(s8_t, [sidx, hsplat])
        gd = plsc.load_gather(s8_t, [didx, hsplat + HEADS])
        r = gs + gd
        r = jnp.where(r >= 0.0, r, r * 0.2) * w
        return jnp.exp(r)

    def p1_compute(k, er):
        for g in range(NG):
            lane = lax.iota(jnp.int32, L) + g * L
            for hl in range(HPC):
                ev = edge_alpha_num(k, g, hl)
                plsc.store_scatter(
                    er, [lane, jnp.full((L,), hl, jnp.int32)], ev)

    def p1_block(b, carry):
        stage(b)

        def p1_chunk(k, c2):
            p1_compute(k, erows0)
            pltpu.sync_copy(erows0, t_sp.at[eb.at[k, 1]], add=True)
            return c2

        lax.fori_loop(0, RPB, p1_chunk, 0)
        return carry

    lax.fori_loop(0, BLOCKS, p1_block, 0)
    plsc.subcore_barrier()

    def p2_subphase(hl):
        row_off = (cid * HPC + hl) * N_NODES

        def issue_gather(k, po):
            for g in range(NG):
                srcadj[po][pl.ds(g * L, L)] = (
                    eb[k, 0, pl.ds(g * L, L)] + row_off)
            return pltpu.async_copy(h4_hbm.at[srcadj[po]], hbuf[po], gsem[po])

        def p2_process(k, po):
            hb = hbuf[po]
            for g in range(NG):
                lane = lax.iota(jnp.int32, L) + g * L
                ev = edge_alpha_num(k, g, hl)
                tg = plsc.load_gather(
                    tbuf[po], [lane, jnp.full((L,), 0, jnp.int32) + hl])
                a = ev / (tg + 1e-8)
                for e16 in range(L):
                    e = g * L + e16
                    aa = a[e16]
                    hb[e, pl.ds(0, L)] = hb[e, pl.ds(0, L)] * aa
                    hb[e, pl.ds(L, L)] = hb[e, pl.ds(L, L)] * aa
            pltpu.sync_copy(hb, out_sp.at[eb.at[k, 1]], add=True)

        def p2_block(b, c2):
            stage(b)

            def p2_pair(m, c3):
                dh0 = issue_gather(2 * m, 0)
                dh1 = issue_gather(2 * m + 1, 1)
                pltpu.sync_copy(t_sp.at[eb.at[2 * m, 1]], tbuf[0])
                dh0.wait()
                p2_process(2 * m, 0)
                pltpu.sync_copy(t_sp.at[eb.at[2 * m + 1, 1]], tbuf[1])
                dh1.wait()
                p2_process(2 * m + 1, 1)
                return c3

            lax.fori_loop(0, RPB // 2, p2_pair, 0)
            return c2

        lax.fori_loop(0, BLOCKS, p2_block, 0)
        plsc.subcore_barrier()
        pltpu.sync_copy(
            out_sp.at[pl.ds(sid * NODE_SLICE, NODE_SLICE)],
            outp_hbm.at[cid, hl, pl.ds(sid * NODE_SLICE, NODE_SLICE)])
        if hl + 1 < HPC:
            pltpu.sync_copy(
                zo_hbm, out_sp.at[pl.ds(sid * NODE_SLICE, NODE_SLICE)])
            plsc.subcore_barrier()

    for hl in range(HPC):
        p2_subphase(hl)


_sc_edge_phase = pl.kernel(
    _sc_body,
    out_type=jax.ShapeDtypeStruct((NC, HPC, N_NODES, OUT_CH), jnp.float32),
    mesh=plsc.VectorSubcoreMesh(core_axis_name="c", subcore_axis_name="s"),
    compiler_params=pltpu.CompilerParams(
        use_tc_tiling_on_sc=False, needs_layout_passes=False),
    scratch_types=[
        pltpu.VMEM((N_NODES, 2 * HEADS), jnp.float32),
        pltpu.VMEM((RPB, 3, CHUNK), jnp.int32),
        pltpu.VMEM((CHUNK,), jnp.int32),
        pltpu.VMEM((CHUNK,), jnp.int32),
        pltpu.VMEM((CHUNK, TW), jnp.float32),
        pltpu.VMEM((CHUNK, TW), jnp.float32),
        pltpu.VMEM((CHUNK, OUT_CH), jnp.float32),
        pltpu.VMEM((CHUNK, OUT_CH), jnp.float32),
        pltpu.VMEM((CHUNK, TW), jnp.float32),
        pltpu.VMEM((CHUNK, TW), jnp.float32),
        pltpu.VMEM_SHARED((N_NODES, TW), jnp.float32),
        pltpu.VMEM_SHARED((N_NODES, OUT_CH), jnp.float32),
        pltpu.SemaphoreType.DMA,
        pltpu.SemaphoreType.DMA,
        pltpu.SemaphoreType.DMA,
        pltpu.SemaphoreType.DMA,
    ],
)


def _proj_body(x_ref, wt_ref, wa_ref, h_ref, s_ref):
    h_ref[...] = jnp.dot(x_ref[...], wt_ref[0],
                         preferred_element_type=jnp.float32)
    s_ref[...] = jnp.dot(x_ref[...], wa_ref[...],
                         preferred_element_type=jnp.float32)


_PROJ_BLK = 2000


def kernel(x, edge_index, edge_weight, W, a_src, a_dst):
    src = edge_index[0].astype(jnp.int32).reshape(ROWS, CHUNK)
    dst = edge_index[1].astype(jnp.int32).reshape(ROWS, CHUNK)
    ewb = lax.bitcast_convert_type(
        edge_weight.astype(jnp.float32), jnp.int32).reshape(ROWS, CHUNK)
    edges = jnp.stack([src, dst, ewb], axis=1)
    wt = W.T.astype(jnp.float32)

    k = jnp.arange(HC)
    head_mask = (k[:, None] // OUT_CH == jnp.arange(HEADS)[None, :])
    a_mat = jnp.concatenate(
        [jnp.where(head_mask, a_src.reshape(-1)[:, None], 0.0),
         jnp.where(head_mask, a_dst.reshape(-1)[:, None], 0.0)],
        axis=1).astype(jnp.float32)
    wa = wt @ a_mat

    zo = jnp.zeros((NODE_SLICE, OUT_CH), jnp.float32)
    zt = jnp.zeros((NODE_SLICE, TW), jnp.float32)

    n_blocks = N_NODES // _PROJ_BLK
    h4, s8 = pl.pallas_call(
        _proj_body,
        grid=(HEADS, n_blocks),
        in_specs=[
            pl.BlockSpec((_PROJ_BLK, IN_CH), lambda hh, i: (i, 0)),
            pl.BlockSpec((1, IN_CH, OUT_CH), lambda hh, i: (hh, 0, 0)),
            pl.BlockSpec((IN_CH, 2 * HEADS), lambda hh, i: (0, 0)),
        ],
        out_specs=[
            pl.BlockSpec((_PROJ_BLK, OUT_CH),
                         lambda hh, i: (hh * (N_NODES // _PROJ_BLK) + i, 0)),
            pl.BlockSpec((_PROJ_BLK, 2 * HEADS), lambda hh, i: (i, 0)),
        ],
        out_shape=[
            jax.ShapeDtypeStruct((HEADS * N_NODES, OUT_CH), jnp.float32),
            jax.ShapeDtypeStruct((N_NODES, 2 * HEADS), jnp.float32),
        ],
    )(x, wt.reshape(IN_CH, HEADS, OUT_CH).transpose(1, 0, 2), wa)

    outp = _sc_edge_phase(h4, s8, edges, zo, zt)
    return jnp.moveaxis(outp.reshape(HEADS, N_NODES, OUT_CH), 0, 1).reshape(
        N_NODES, HC)


# speedup vs baseline: 64.0533x; 1.0790x over previous
import jax
import jax.numpy as jnp
from jax import lax
from jax.experimental import pallas as pl
from jax.experimental.pallas import tpu as pltpu
from jax.experimental.pallas import tpu_sc as plsc

HEADS = 4
OUT_CH = 32
IN_CH = 128
N_NODES = 10000
N_EDGES = 320000
HC = HEADS * OUT_CH

NC = 2
NS = 16
L = 16
HPC = HEADS // NC
TW = 8

CHUNK = 80
NG = CHUNK // L
ROWS = N_EDGES // CHUNK

ROWS_PER_TILE = N_EDGES // NS // CHUNK
BLOCKS = 5
RPB = ROWS_PER_TILE // BLOCKS

NODE_SLICE = N_NODES // NS


def _sc_body(h4_hbm, s8_hbm, edges_hbm, zo_hbm, zt_hbm,
             outp_hbm, s8_t, eb,
             srcadj0, srcadj1, erows0, erows1, hbuf0, hbuf1, tbuf0, tbuf1,
             t_sp, out_sp, gsem0, gsem1, ssem0, ssem1):
    cid = lax.axis_index("c")
    sid = lax.axis_index("s")
    gsem = [gsem0, gsem1]
    ssem = [ssem0, ssem1]
    erows = [erows0, erows1]
    hbuf = [hbuf0, hbuf1]
    tbuf = [tbuf0, tbuf1]
    srcadj = [srcadj0, srcadj1]

    pltpu.sync_copy(s8_hbm, s8_t)
    pltpu.sync_copy(zo_hbm, out_sp.at[pl.ds(sid * NODE_SLICE, NODE_SLICE)])
    pltpu.sync_copy(zt_hbm, t_sp.at[pl.ds(sid * NODE_SLICE, NODE_SLICE)])
    pltpu.sync_copy(zt_hbm.at[pl.ds(0, CHUNK)], erows0)
    pltpu.sync_copy(zt_hbm.at[pl.ds(0, CHUNK)], erows1)
    plsc.subcore_barrier()

    def stage(b):
        base = sid * ROWS_PER_TILE + b * RPB
        pltpu.sync_copy(edges_hbm.at[pl.ds(base, RPB)], eb)

    def edge_vecs(k, g):
        sidx = eb[k, 0, pl.ds(g * L, L)]
        didx = eb[k, 1, pl.ds(g * L, L)]
        w = plsc.bitcast(eb[k, 2, pl.ds(g * L, L)], jnp.float32)
        return sidx, didx, w

    def edge_alpha_num(k, g, hl):
        sidx, didx, w = edge_vecs(k, g)
        hsplat = jnp.full((L,), 0, jnp.int32) + (hl + cid * HPC)
        gs = plsc.load_gather(s8_t, [sidx, hsplat])
        gd = plsc.load_gather(s8_t, [didx, hsplat + HEADS])
        r = gs + gd
        r = jnp.where(r >= 0.0, r, r * 0.2) * w
        return jnp.exp(r)

    def p1_compute(k, er):
        for g in range(NG):
            lane = lax.iota(jnp.int32, L) + g * L
            for hl in range(HPC):
                ev = edge_alpha_num(k, g, hl)
                plsc.store_scatter(
                    er, [lane, jnp.full((L,), hl, jnp.int32)], ev)

    def p1_block(b, carry):
        stage(b)

        def p1_pair(m, c2):
            p1_compute(2 * m, erows0)
            da0 = pltpu.async_copy(
                erows0, t_sp.at[eb.at[2 * m, 1]], ssem[0], add=True)
            p1_compute(2 * m + 1, erows1)
            da1 = pltpu.async_copy(
                erows1, t_sp.at[eb.at[2 * m + 1, 1]], ssem[1], add=True)
            da0.wait()
            da1.wait()
            return c2

        lax.fori_loop(0, RPB // 2, p1_pair, 0)
        return carry

    lax.fori_loop(0, BLOCKS, p1_block, 0)
    plsc.subcore_barrier()

    def p2_subphase(hl):
        row_off = (cid * HPC + hl) * N_NODES

        def issue_gather(k, po):
            for g in range(NG):
                srcadj[po][pl.ds(g * L, L)] = (
                    eb[k, 0, pl.ds(g * L, L)] + row_off)
            return pltpu.async_copy(h4_hbm.at[srcadj[po]], hbuf[po], gsem[po])

        def p2_process(k, po):
            hb = hbuf[po]
            for g in range(NG):
                lane = lax.iota(jnp.int32, L) + g * L
                ev = edge_alpha_num(k, g, hl)
                tg = plsc.load_gather(
                    tbuf[po], [lane, jnp.full((L,), 0, jnp.int32) + hl])
                a = ev / (tg + 1e-8)
                for e16 in range(L):
                    e = g * L + e16
                    aa = a[e16]
                    hb[e, pl.ds(0, L)] = hb[e, pl.ds(0, L)] * aa
                    hb[e, pl.ds(L, L)] = hb[e, pl.ds(L, L)] * aa
            return pltpu.async_copy(
                hb, out_sp.at[eb.at[k, 1]], ssem[po], add=True)

        def p2_block(b, c2):
            stage(b)

            def p2_pair(m, c3):
                dh0 = issue_gather(2 * m, 0)
                dh1 = issue_gather(2 * m + 1, 1)
                pltpu.sync_copy(t_sp.at[eb.at[2 * m, 1]], tbuf[0])
                dh0.wait()
                ds0 = p2_process(2 * m, 0)
                pltpu.sync_copy(t_sp.at[eb.at[2 * m + 1, 1]], tbuf[1])
                dh1.wait()
                ds1 = p2_process(2 * m + 1, 1)
                ds0.wait()
                ds1.wait()
                return c3

            lax.fori_loop(0, RPB // 2, p2_pair, 0)
            return c2

        lax.fori_loop(0, BLOCKS, p2_block, 0)
        plsc.subcore_barrier()
        pltpu.sync_copy(
            out_sp.at[pl.ds(sid * NODE_SLICE, NODE_SLICE)],
            outp_hbm.at[cid, hl, pl.ds(sid * NODE_SLICE, NODE_SLICE)])
        if hl + 1 < HPC:
            pltpu.sync_copy(
                zo_hbm, out_sp.at[pl.ds(sid * NODE_SLICE, NODE_SLICE)])
            plsc.subcore_barrier()

    for hl in range(HPC):
        p2_subphase(hl)


_sc_edge_phase = pl.kernel(
    _sc_body,
    out_type=jax.ShapeDtypeStruct((NC, HPC, N_NODES, OUT_CH), jnp.float32),
    mesh=plsc.VectorSubcoreMesh(core_axis_name="c", subcore_axis_name="s"),
    compiler_params=pltpu.CompilerParams(
        use_tc_tiling_on_sc=False, needs_layout_passes=False),
    scratch_types=[
        pltpu.VMEM((N_NODES, 2 * HEADS), jnp.float32),
        pltpu.VMEM((RPB, 3, CHUNK), jnp.int32),
        pltpu.VMEM((CHUNK,), jnp.int32),
        pltpu.VMEM((CHUNK,), jnp.int32),
        pltpu.VMEM((CHUNK, TW), jnp.float32),
        pltpu.VMEM((CHUNK, TW), jnp.float32),
        pltpu.VMEM((CHUNK, OUT_CH), jnp.float32),
        pltpu.VMEM((CHUNK, OUT_CH), jnp.float32),
        pltpu.VMEM((CHUNK, TW), jnp.float32),
        pltpu.VMEM((CHUNK, TW), jnp.float32),
        pltpu.VMEM_SHARED((N_NODES, TW), jnp.float32),
        pltpu.VMEM_SHARED((N_NODES, OUT_CH), jnp.float32),
        pltpu.SemaphoreType.DMA,
        pltpu.SemaphoreType.DMA,
        pltpu.SemaphoreType.DMA,
        pltpu.SemaphoreType.DMA,
    ],
)


def _proj_body(x_ref, wt_ref, wa_ref, h_ref, s_ref):
    h_ref[...] = jnp.dot(x_ref[...], wt_ref[0],
                         preferred_element_type=jnp.float32)
    s_ref[...] = jnp.dot(x_ref[...], wa_ref[...],
                         preferred_element_type=jnp.float32)


_PROJ_BLK = 2000


def kernel(x, edge_index, edge_weight, W, a_src, a_dst):
    src = edge_index[0].astype(jnp.int32).reshape(ROWS, CHUNK)
    dst = edge_index[1].astype(jnp.int32).reshape(ROWS, CHUNK)
    ewb = lax.bitcast_convert_type(
        edge_weight.astype(jnp.float32), jnp.int32).reshape(ROWS, CHUNK)
    edges = jnp.stack([src, dst, ewb], axis=1)
    wt = W.T.astype(jnp.float32)

    k = jnp.arange(HC)
    head_mask = (k[:, None] // OUT_CH == jnp.arange(HEADS)[None, :])
    a_mat = jnp.concatenate(
        [jnp.where(head_mask, a_src.reshape(-1)[:, None], 0.0),
         jnp.where(head_mask, a_dst.reshape(-1)[:, None], 0.0)],
        axis=1).astype(jnp.float32)
    wa = wt @ a_mat

    zo = jnp.zeros((NODE_SLICE, OUT_CH), jnp.float32)
    zt = jnp.zeros((NODE_SLICE, TW), jnp.float32)

    n_blocks = N_NODES // _PROJ_BLK
    h4, s8 = pl.pallas_call(
        _proj_body,
        grid=(HEADS, n_blocks),
        in_specs=[
            pl.BlockSpec((_PROJ_BLK, IN_CH), lambda hh, i: (i, 0)),
            pl.BlockSpec((1, IN_CH, OUT_CH), lambda hh, i: (hh, 0, 0)),
            pl.BlockSpec((IN_CH, 2 * HEADS), lambda hh, i: (0, 0)),
        ],
        out_specs=[
            pl.BlockSpec((_PROJ_BLK, OUT_CH),
                         lambda hh, i: (hh * (N_NODES // _PROJ_BLK) + i, 0)),
            pl.BlockSpec((_PROJ_BLK, 2 * HEADS), lambda hh, i: (i, 0)),
        ],
        out_shape=[
            jax.ShapeDtypeStruct((HEADS * N_NODES, OUT_CH), jnp.float32),
            jax.ShapeDtypeStruct((N_NODES, 2 * HEADS), jnp.float32),
        ],
    )(x, wt.reshape(IN_CH, HEADS, OUT_CH).transpose(1, 0, 2), wa)

    outp = _sc_edge_phase(h4, s8, edges, zo, zt)
    return jnp.moveaxis(outp.reshape(HEADS, N_NODES, OUT_CH), 0, 1).reshape(
        N_NODES, HC)


# speedup vs baseline: 65.6820x; 1.0254x over previous
import jax
import jax.numpy as jnp
from jax import lax
from jax.experimental import pallas as pl
from jax.experimental.pallas import tpu as pltpu
from jax.experimental.pallas import tpu_sc as plsc

HEADS = 4
OUT_CH = 32
IN_CH = 128
N_NODES = 10000
N_EDGES = 320000
HC = HEADS * OUT_CH

NC = 2
NS = 16
L = 16
HPC = HEADS // NC
TW = 8

CHUNK = 80
NG = CHUNK // L
ROWS = N_EDGES // CHUNK

ROWS_PER_TILE = N_EDGES // NS // CHUNK
BLOCKS = 10
RPB = ROWS_PER_TILE // BLOCKS
NB = 5

NODE_SLICE = N_NODES // NS


def _sc_body(h4_hbm, s8_hbm, edges_hbm, zo_hbm, zt_hbm,
             outp_hbm, s8_t, eb,
             srcadj0, srcadj1, srcadj2, srcadj3, srcadj4,
             erows0, erows1,
             hbuf0, hbuf1, hbuf2, hbuf3, hbuf4,
             tbuf0, tbuf1, tbuf2, tbuf3, tbuf4,
             t_sp, out_sp, gsem0, gsem1, gsem2, gsem3, gsem4, ssem0, ssem1):
    cid = lax.axis_index("c")
    sid = lax.axis_index("s")
    gsem = [gsem0, gsem1, gsem2, gsem3, gsem4]
    ssem = [ssem0, ssem1]
    erows = [erows0, erows1]
    hbuf = [hbuf0, hbuf1, hbuf2, hbuf3, hbuf4]
    tbuf = [tbuf0, tbuf1, tbuf2, tbuf3, tbuf4]
    srcadj = [srcadj0, srcadj1, srcadj2, srcadj3, srcadj4]

    pltpu.sync_copy(s8_hbm, s8_t)
    pltpu.sync_copy(zo_hbm, out_sp.at[pl.ds(sid * NODE_SLICE, NODE_SLICE)])
    pltpu.sync_copy(zt_hbm, t_sp.at[pl.ds(sid * NODE_SLICE, NODE_SLICE)])
    pltpu.sync_copy(zt_hbm.at[pl.ds(0, CHUNK)], erows0)
    pltpu.sync_copy(zt_hbm.at[pl.ds(0, CHUNK)], erows1)
    plsc.subcore_barrier()

    def stage(b):
        base = sid * ROWS_PER_TILE + b * RPB
        pltpu.sync_copy(edges_hbm.at[pl.ds(base, RPB)], eb)

    def edge_vecs(k, g):
        sidx = eb[k, 0, pl.ds(g * L, L)]
        didx = eb[k, 1, pl.ds(g * L, L)]
        w = plsc.bitcast(eb[k, 2, pl.ds(g * L, L)], jnp.float32)
        return sidx, didx, w

    def edge_alpha_num(k, g, hl):
        sidx, didx, w = edge_vecs(k, g)
        hsplat = jnp.full((L,), 0, jnp.int32) + (hl + cid * HPC)
        gs = plsc.load_gather(s8_t, [sidx, hsplat])
        gd = plsc.load_gather(s8_t, [didx, hsplat + HEADS])
        r = gs + gd
        r = jnp.where(r >= 0.0, r, r * 0.2) * w
        return jnp.exp(r)

    def p1_compute(k, er):
        for g in range(NG):
            lane = lax.iota(jnp.int32, L) + g * L
            for hl in range(HPC):
                ev = edge_alpha_num(k, g, hl)
                plsc.store_scatter(
                    er, [lane, jnp.full((L,), hl, jnp.int32)], ev)

    def p1_block(b, carry):
        stage(b)

        def p1_pair(m, c2):
            p1_compute(2 * m, erows0)
            da0 = pltpu.async_copy(
                erows0, t_sp.at[eb.at[2 * m, 1]], ssem[0], add=True)
            p1_compute(2 * m + 1, erows1)
            da1 = pltpu.async_copy(
                erows1, t_sp.at[eb.at[2 * m + 1, 1]], ssem[1], add=True)
            da0.wait()
            da1.wait()
            return c2

        lax.fori_loop(0, RPB // 2, p1_pair, 0)
        if RPB % 2:
            p1_compute(RPB - 1, erows0)
            pltpu.async_copy(
                erows0, t_sp.at[eb.at[RPB - 1, 1]], ssem[0], add=True).wait()
        return carry

    lax.fori_loop(0, BLOCKS, p1_block, 0)
    plsc.subcore_barrier()

    def p2_subphase(hl):
        row_off = (cid * HPC + hl) * N_NODES

        def issue_gather(k, po):
            for g in range(NG):
                srcadj[po][pl.ds(g * L, L)] = (
                    eb[k, 0, pl.ds(g * L, L)] + row_off)
            return pltpu.async_copy(h4_hbm.at[srcadj[po]], hbuf[po], gsem[po])

        def p2_process(k, po):
            hb = hbuf[po]
            for g in range(NG):
                lane = lax.iota(jnp.int32, L) + g * L
                ev = edge_alpha_num(k, g, hl)
                tg = plsc.load_gather(
                    tbuf[po], [lane, jnp.full((L,), 0, jnp.int32) + hl])
                a = ev / (tg + 1e-8)
                for e16 in range(L):
                    e = g * L + e16
                    aa = a[e16]
                    hb[e, pl.ds(0, L)] = hb[e, pl.ds(0, L)] * aa
                    hb[e, pl.ds(L, L)] = hb[e, pl.ds(L, L)] * aa
            return pltpu.async_copy(
                hb, out_sp.at[eb.at[k, 1]], ssem[0], add=True)

        def p2_block(b, c2):
            stage(b)

            def p2_round(m, c3):
                dhs = [issue_gather(NB * m + j, j) for j in range(NB)]
                dss = []
                for j in range(NB):
                    k = NB * m + j
                    pltpu.sync_copy(t_sp.at[eb.at[k, 1]], tbuf[j])
                    dhs[j].wait()
                    dss.append(p2_process(k, j))
                for ds in dss:
                    ds.wait()
                return c3

            lax.fori_loop(0, RPB // NB, p2_round, 0)
            return c2

        lax.fori_loop(0, BLOCKS, p2_block, 0)
        plsc.subcore_barrier()
        pltpu.sync_copy(
            out_sp.at[pl.ds(sid * NODE_SLICE, NODE_SLICE)],
            outp_hbm.at[cid, hl, pl.ds(sid * NODE_SLICE, NODE_SLICE)])
        if hl + 1 < HPC:
            pltpu.sync_copy(
                zo_hbm, out_sp.at[pl.ds(sid * NODE_SLICE, NODE_SLICE)])
            plsc.subcore_barrier()

    for hl in range(HPC):
        p2_subphase(hl)


_sc_edge_phase = pl.kernel(
    _sc_body,
    out_type=jax.ShapeDtypeStruct((NC, HPC, N_NODES, OUT_CH), jnp.float32),
    mesh=plsc.VectorSubcoreMesh(core_axis_name="c", subcore_axis_name="s"),
    compiler_params=pltpu.CompilerParams(
        use_tc_tiling_on_sc=False, needs_layout_passes=False),
    scratch_types=[
        pltpu.VMEM((N_NODES, 2 * HEADS), jnp.float32),
        pltpu.VMEM((RPB, 3, CHUNK), jnp.int32),
        pltpu.VMEM((CHUNK,), jnp.int32),
        pltpu.VMEM((CHUNK,), jnp.int32),
        pltpu.VMEM((CHUNK,), jnp.int32),
        pltpu.VMEM((CHUNK,), jnp.int32),
        pltpu.VMEM((CHUNK,), jnp.int32),
        pltpu.VMEM((CHUNK, TW), jnp.float32),
        pltpu.VMEM((CHUNK, TW), jnp.float32),
        pltpu.VMEM((CHUNK, OUT_CH), jnp.float32),
        pltpu.VMEM((CHUNK, OUT_CH), jnp.float32),
        pltpu.VMEM((CHUNK, OUT_CH), jnp.float32),
        pltpu.VMEM((CHUNK, OUT_CH), jnp.float32),
        pltpu.VMEM((CHUNK, OUT_CH), jnp.float32),
        pltpu.VMEM((CHUNK, TW), jnp.float32),
        pltpu.VMEM((CHUNK, TW), jnp.float32),
        pltpu.VMEM((CHUNK, TW), jnp.float32),
        pltpu.VMEM((CHUNK, TW), jnp.float32),
        pltpu.VMEM((CHUNK, TW), jnp.float32),
        pltpu.VMEM_SHARED((N_NODES, TW), jnp.float32),
        pltpu.VMEM_SHARED((N_NODES, OUT_CH), jnp.float32),
        pltpu.SemaphoreType.DMA,
        pltpu.SemaphoreType.DMA,
        pltpu.SemaphoreType.DMA,
        pltpu.SemaphoreType.DMA,
        pltpu.SemaphoreType.DMA,
        pltpu.SemaphoreType.DMA,
        pltpu.SemaphoreType.DMA,
    ],
)


def _proj_body(x_ref, wt_ref, wa_ref, h_ref, s_ref):
    h_ref[...] = jnp.dot(x_ref[...], wt_ref[0],
                         preferred_element_type=jnp.float32)
    s_ref[...] = jnp.dot(x_ref[...], wa_ref[...],
                         preferred_element_type=jnp.float32)


_PROJ_BLK = 2000


def kernel(x, edge_index, edge_weight, W, a_src, a_dst):
    src = edge_index[0].astype(jnp.int32).reshape(ROWS, CHUNK)
    dst = edge_index[1].astype(jnp.int32).reshape(ROWS, CHUNK)
    ewb = lax.bitcast_convert_type(
        edge_weight.astype(jnp.float32), jnp.int32).reshape(ROWS, CHUNK)
    edges = jnp.stack([src, dst, ewb], axis=1)
    wt = W.T.astype(jnp.float32)

    k = jnp.arange(HC)
    head_mask = (k[:, None] // OUT_CH == jnp.arange(HEADS)[None, :])
    a_mat = jnp.concatenate(
        [jnp.where(head_mask, a_src.reshape(-1)[:, None], 0.0),
         jnp.where(head_mask, a_dst.reshape(-1)[:, None], 0.0)],
        axis=1).astype(jnp.float32)
    wa = wt @ a_mat

    zo = jnp.zeros((NODE_SLICE, OUT_CH), jnp.float32)
    zt = jnp.zeros((NODE_SLICE, TW), jnp.float32)

    n_blocks = N_NODES // _PROJ_BLK
    h4, s8 = pl.pallas_call(
        _proj_body,
        grid=(HEADS, n_blocks),
        in_specs=[
            pl.BlockSpec((_PROJ_BLK, IN_CH), lambda hh, i: (i, 0)),
            pl.BlockSpec((1, IN_CH, OUT_CH), lambda hh, i: (hh, 0, 0)),
            pl.BlockSpec((IN_CH, 2 * HEADS), lambda hh, i: (0, 0)),
        ],
        out_specs=[
            pl.BlockSpec((_PROJ_BLK, OUT_CH),
                         lambda hh, i: (hh * (N_NODES // _PROJ_BLK) + i, 0)),
            pl.BlockSpec((_PROJ_BLK, 2 * HEADS), lambda hh, i: (i, 0)),
        ],
        out_shape=[
            jax.ShapeDtypeStruct((HEADS * N_NODES, OUT_CH), jnp.float32),
            jax.ShapeDtypeStruct((N_NODES, 2 * HEADS), jnp.float32),
        ],
    )(x, wt.reshape(IN_CH, HEADS, OUT_CH).transpose(1, 0, 2), wa)

    outp = _sc_edge_phase(h4, s8, edges, zo, zt)
    return jnp.moveaxis(outp.reshape(HEADS, N_NODES, OUT_CH), 0, 1).reshape(
        N_NODES, HC)
